# baseline (speedup 1.0000x reference)
import jax
import jax.numpy as jnp
from jax import lax
from jax.experimental import pallas as pl
from jax.experimental.pallas import tpu as pltpu
from jax.experimental.pallas import tpu_sc as plsc

N_NODES = 4096
N_EDGES = 65536
D_IN = 128
H1 = 128
H2 = 128
NC1, NC2, NC3 = 100, 70, 50
NSUBG = 64
EPS = 1e-15

_L = 16
_NCORE = 2
_NSUB = 16
_NW = _NCORE * _NSUB
_C = 128
_P1W = 128
_DROWS = 32


def _make_edge_accum(d_in, with_w):
    ept = N_EDGES // _NW
    nchunk = ept // _C
    stripe = N_NODES // _NSUB
    mesh = plsc.VectorSubcoreMesh(core_axis_name="c", subcore_axis_name="s")

    nrow = ept // _C

    def body(src_hbm, dst_hbm, ew_hbm, val_hbm, *rest):
        if with_w:
            (out_hbm, outdeg_hbm, src_v, dst_v, w_v, g0_v, g1_v, g2_v,
             deg_v, acc, sem_g, sem_s) = rest
        else:
            (out_hbm, src_v, dst_v, w_v, g0_v, g1_v, g2_v, acc,
             sem_g, sem_s) = rest
        gbufs = (g0_v, g1_v, g2_v)
        f0_v = g0_v
        cid = lax.axis_index("c")
        sid = lax.axis_index("s")
        wid = cid * _NSUB + sid
        zeros16 = jnp.zeros((_L,), jnp.float32)
        lane = lax.broadcasted_iota(jnp.int32, (_L,), 0)
        mask0 = lane == 0

        pltpu.sync_copy(src_hbm.at[pl.ds(wid * nrow, nrow)], src_v)
        pltpu.sync_copy(dst_hbm.at[pl.ds(wid * nrow, nrow)], dst_v)
        pltpu.sync_copy(ew_hbm.at[pl.ds(wid * nrow, nrow)], w_v)

        def zero_row(j, carry):
            for q in range(d_in // _L):
                f0_v[j, pl.ds(q * _L, _L)] = zeros16
            return carry

        lax.fori_loop(0, _C, zero_row, 0)
        for t in range(stripe // _C):
            pltpu.sync_copy(f0_v,
                            acc.at[pl.ds(sid * stripe + t * _C, _C)])
        if with_w:
            def zero_deg(j, carry):
                for q in range(128 // _L):
                    deg_v[j, pl.ds(q * _L, _L)] = zeros16
                return carry

            lax.fori_loop(0, _DROWS, zero_deg, 0)

        nbuf = 3
        for j in range(min(nbuf, nchunk)):
            pltpu.async_copy(val_hbm.at[dst_v.at[j]], gbufs[j], sem_g.at[j])
        plsc.subcore_barrier()

        for kk in range(nchunk):
            b = kk % nbuf
            gbuf = gbufs[b]
            fbuf = gbuf
            pltpu.make_async_copy(val_hbm.at[dst_v.at[kk]], gbuf,
                                  sem_g.at[b]).wait()
            if kk >= 1 and kk + nbuf - 1 < nchunk:
                bb = (kk + nbuf - 1) % nbuf
                pltpu.make_async_copy(gbufs[bb], acc.at[src_v.at[kk - 1]],
                                      sem_s.at[bb]).wait()
                pltpu.async_copy(val_hbm.at[dst_v.at[kk + nbuf - 1]],
                                 gbufs[bb], sem_g.at[bb])

            kb = jnp.full((_L,), kk, jnp.int32)

            @plsc.parallel_loop(0, _C, unroll=8)
            def scale_row(e):
                eb = jnp.full((_L,), e, jnp.int32)
                wbc = plsc.load_gather(w_v, [kb, eb])
                for q in range(d_in // _L):
                    fbuf[e, pl.ds(q * _L, _L)] = (
                        gbuf[e, pl.ds(q * _L, _L)] * wbc)
                if with_w:
                    sbc = plsc.load_gather(src_v, [kb, eb])
                    plsc.addupdate_scatter(
                        deg_v,
                        [lax.shift_right_logical(sbc, 7),
                         jnp.bitwise_and(sbc, 127)],
                        wbc, mask=mask0)
            pltpu.async_copy(fbuf, acc.at[src_v.at[kk]], sem_s.at[b],
                             add=True)

        for j in range(max(nchunk - nbuf, 1), nchunk):
            pltpu.make_async_copy(gbufs[j % nbuf], acc.at[src_v.at[j]],
                                  sem_s.at[j % nbuf]).wait()
        plsc.subcore_barrier()
        for t in range(stripe // _C):
            r0 = sid * stripe + t * _C
            pltpu.sync_copy(acc.at[pl.ds(r0, _C)],
                            out_hbm.at[pl.ds(cid * N_NODES + r0, _C)])
        if with_w:
            pltpu.sync_copy(deg_v, outdeg_hbm.at[pl.ds(wid * _DROWS, _DROWS)])

    out_type = [jax.ShapeDtypeStruct((_NCORE * N_NODES, d_in), jnp.float32)]
    scratch = [
        pltpu.VMEM((nrow, _C), jnp.int32),
        pltpu.VMEM((nrow, _C), jnp.int32),
        pltpu.VMEM((nrow, _C), jnp.float32),
        pltpu.VMEM((_C, d_in), jnp.float32),
        pltpu.VMEM((_C, d_in), jnp.float32),
        pltpu.VMEM((_C, d_in), jnp.float32),
    ]
    if with_w:
        out_type.append(
            jax.ShapeDtypeStruct((_NW * _DROWS, 128), jnp.float32))
        scratch.append(pltpu.VMEM((_DROWS, 128), jnp.float32))
    scratch += [
        pltpu.VMEM_SHARED((N_NODES, d_in), jnp.float32),
        pltpu.SemaphoreType.DMA((3,)),
        pltpu.SemaphoreType.DMA((3,)),
    ]
    return pl.kernel(
        body,
        mesh=mesh,
        compiler_params=pltpu.CompilerParams(needs_layout_passes=False),
        out_type=out_type,
        scratch_types=scratch,
    )


def _tc1_body(x_ref, emb_ref, gnw_ref, gnb_ref, gna_ref, w1_ref, b1_ref,
              out_ref):
    xv = x_ref[...]
    col = lax.broadcasted_iota(jnp.int32, (N_NODES, D_IN), 1)
    onehot = (col == xv).astype(jnp.float32)
    x1 = jnp.dot(onehot, emb_ref[...], preferred_element_type=jnp.float32)
    mean = jnp.mean(x1, axis=0, keepdims=True)
    cent = x1 - gna_ref[...] * mean
    var = jnp.mean(cent * cent, axis=0, keepdims=True)
    x1n = gnw_ref[...] * cent / jnp.sqrt(var + 1e-5) + gnb_ref[...]
    h = jnp.dot(x1n, w1_ref[...], preferred_element_type=jnp.float32)
    out_ref[...] = jnp.maximum(h + b1_ref[...], 0.0)


def _scal(v):
    return jnp.full((1, 1), v, jnp.float32)


def _tc2_body(acc_ref, deg_ref, sa_ref, m1w_ref, m1b_ref, bnw_ref,
              bnb_ref, p1_ref, out1_ref, sc1t_ref, misc_ref):
    swh = acc_ref[0:N_NODES, :] + acc_ref[N_NODES:2 * N_NODES, :]
    degsum = deg_ref[0:_DROWS, :]
    for t in range(1, _NW):
        degsum = degsum + deg_ref[t * _DROWS:(t + 1) * _DROWS, :]
    ri32 = lax.broadcasted_iota(jnp.int32, (N_NODES, _DROWS), 0)
    ci32 = lax.broadcasted_iota(jnp.int32, (N_NODES, _DROWS), 1)
    a1 = (ci32 == lax.shift_right_logical(ri32, 7)).astype(jnp.float32)
    t128 = jnp.dot(a1, degsum, preferred_element_type=jnp.float32)
    ri = lax.broadcasted_iota(jnp.int32, (N_NODES, 128), 0)
    ci = lax.broadcasted_iota(jnp.int32, (N_NODES, 128), 1)
    msel = (ci == jnp.bitwise_and(ri, 127)).astype(jnp.float32)
    deg = jnp.sum(t128 * msel, axis=1, keepdims=True)
    degc = jnp.where(deg < 0.5, deg + 1.0, deg)
    conv = swh / degc
    mean = jnp.mean(conv, axis=0, keepdims=True)
    var = jnp.mean(conv * conv, axis=0, keepdims=True) - mean * mean
    h = jnp.maximum(
        (conv - mean) / jnp.sqrt(var + 1e-5) * bnw_ref[...] + bnb_ref[...],
        0.0)
    s1 = jnp.dot(h, m1w_ref[...], preferred_element_type=jnp.float32) \
        + m1b_ref[...]
    mx = jnp.max(s1, axis=1, keepdims=True)
    ex = jnp.exp(s1 - mx)
    p1 = ex / jnp.sum(ex, axis=1, keepdims=True)
    ent1 = jnp.sum(-p1 * jnp.log(p1 + EPS)) / N_NODES
    csum = jnp.sum(p1, axis=0, keepdims=True)
    inv = 1.0 / jnp.maximum(csum, 1e-12)
    sc1t = lax.dot_general(sa_ref[...], p1,
                           (((1,), (0,)), ((), ()))) * inv
    out1 = lax.dot_general(p1, h, (((0,), (0,)), ((), ())))
    ss = lax.dot_general(p1, p1, (((0,), (0,)), ((), ())))
    ssf = jnp.sqrt(jnp.sum(ss * ss))
    r_i = lax.broadcasted_iota(jnp.int32, (NC1, NC1), 0)
    c_i = lax.broadcasted_iota(jnp.int32, (NC1, NC1), 1)
    eye1 = (r_i == c_i).astype(jnp.float32)
    o1 = jnp.sqrt(jnp.sum((ss / ssf - eye1 * (1.0 / jnp.sqrt(float(NC1))))**2))
    den1 = jnp.sum(deg * jnp.sum(p1 * p1, axis=1, keepdims=True))
    p1_ref[...] = jnp.concatenate(
        [p1, jnp.zeros((N_NODES, _P1W - NC1), jnp.float32)], axis=1)
    out1_ref[...] = out1
    sc1t_ref[...] = sc1t
    misc_ref[...] = jnp.concatenate(
        [_scal(ent1), _scal(den1), _scal(o1),
         jnp.zeros((1, 125), jnp.float32)], axis=1)


def _eye(n):
    r_i = lax.broadcasted_iota(jnp.int32, (n, n), 0)
    c_i = lax.broadcasted_iota(jnp.int32, (n, n), 1)
    return (r_i == c_i).astype(jnp.float32)


def _row_of(colvec, eye):
    return lax.dot_general(colvec, eye, (((0,), (0,)), ((), ())))


def _topk5(m, ncols):
    iot = lax.broadcasted_iota(jnp.int32, m.shape, 1)
    cols = []
    cur = m
    for _ in range(5):
        mxv = jnp.max(cur, axis=1, keepdims=True)
        cols.append(mxv)
        pos = jnp.min(jnp.where(cur == mxv, iot, ncols), axis=1,
                      keepdims=True)
        cur = jnp.where(iot == pos, -1e30, cur)
    return jnp.concatenate(
        cols + [jnp.zeros((m.shape[0], 3), jnp.float32)], axis=1)


def _tc3_body(acc_ref, p1_ref, out1_ref, sc1t_ref, misc_ref,
              w2_ref, b2_ref, bn1w_ref, bn1b_ref, m2w_ref, m2b_ref,
              w3_ref, b3_ref, bn2w_ref, bn2b_ref, m3w_ref, m3b_ref,
              e1_ref, e2_ref, scal_ref):
    As = acc_ref[0:N_NODES, :] + acc_ref[N_NODES:2 * N_NODES, :]
    p1 = p1_ref[...]
    oadj_t = lax.dot_general(p1, As, (((0,), (0,)), ((), ())))
    oadj = oadj_t[0:NC1, 0:NC1]
    ent1 = misc_ref[0, 0]
    den1 = misc_ref[0, 1]
    o1 = misc_ref[0, 2]
    eye1 = _eye(NC1)
    mc1 = -(jnp.sum(oadj * eye1) / den1)
    oadj = oadj * (1.0 - eye1)
    d1 = jnp.sqrt(jnp.sum(oadj, axis=1, keepdims=True)) + EPS
    oadj1 = oadj / d1 / _row_of(d1, eye1)

    deg2 = jnp.sum(oadj1, axis=1, keepdims=True)
    deg2c = jnp.where(deg2 < 0.5, deg2 + 1.0, deg2)
    hh = jnp.maximum(
        jnp.dot(out1_ref[...], w2_ref[...],
                preferred_element_type=jnp.float32) + b2_ref[...], 0.0)
    h2c = jnp.dot(oadj1 / deg2c, hh, preferred_element_type=jnp.float32)
    m2 = jnp.mean(h2c, axis=0, keepdims=True)
    v2 = jnp.mean(h2c * h2c, axis=0, keepdims=True) - m2 * m2
    h2 = jnp.maximum(
        (h2c - m2) / jnp.sqrt(v2 + 1e-5) * bn1w_ref[...] + bn1b_ref[...], 0.0)
    s2 = jnp.dot(h2, m2w_ref[...], preferred_element_type=jnp.float32) \
        + m2b_ref[...]
    mx2 = jnp.max(s2, axis=1, keepdims=True)
    ex2 = jnp.exp(s2 - mx2)
    p2 = ex2 / jnp.sum(ex2, axis=1, keepdims=True)
    ent2 = jnp.sum(-p2 * jnp.log(p2 + EPS)) / NC1
    cs2 = jnp.sum(p2, axis=0, keepdims=True)
    sc2t = lax.dot_general(sc1t_ref[...], p2, (((1,), (0,)), ((), ()))) \
        * (1.0 / jnp.maximum(cs2, 1e-12))

    out2 = lax.dot_general(p2, h2, (((0,), (0,)), ((), ())))
    ap2 = jnp.dot(oadj1, p2, preferred_element_type=jnp.float32)
    oadj2t = lax.dot_general(p2, ap2, (((0,), (0,)), ((), ())))
    den2 = jnp.sum(deg2 * jnp.sum(p2 * p2, axis=1, keepdims=True))
    eye2 = _eye(NC2)
    mc2 = -(jnp.sum(oadj2t * eye2) / den2)
    ss2 = lax.dot_general(p2, p2, (((0,), (0,)), ((), ())))
    ssf2 = jnp.sqrt(jnp.sum(ss2 * ss2))
    o2 = jnp.sqrt(jnp.sum(
        (ss2 / ssf2 - eye2 * (1.0 / jnp.sqrt(float(NC2))))**2))
    oadj2 = oadj2t * (1.0 - eye2)
    dd = jnp.sqrt(jnp.sum(oadj2, axis=1, keepdims=True)) + EPS
    oadj2n = oadj2 / dd / _row_of(dd, eye2)

    deg3 = jnp.sum(oadj2n, axis=1, keepdims=True)
    deg3c = jnp.where(deg3 < 0.5, deg3 + 1.0, deg3)
    hh3 = jnp.maximum(
        jnp.dot(out2, w3_ref[...],
                preferred_element_type=jnp.float32) + b3_ref[...], 0.0)
    h3c = jnp.dot(oadj2n / deg3c, hh3, preferred_element_type=jnp.float32)
    m3 = jnp.mean(h3c, axis=0, keepdims=True)
    v3 = jnp.mean(h3c * h3c, axis=0, keepdims=True) - m3 * m3
    h3 = jnp.maximum(
        (h3c - m3) / jnp.sqrt(v3 + 1e-5) * bn2w_ref[...] + bn2b_ref[...], 0.0)
    s3 = jnp.dot(h3, m3w_ref[...], preferred_element_type=jnp.float32) \
        + m3b_ref[...]
    mx3 = jnp.max(s3, axis=1, keepdims=True)
    ex3 = jnp.exp(s3 - mx3)
    p3 = ex3 / jnp.sum(ex3, axis=1, keepdims=True)
    ent3 = jnp.sum(-p3 * jnp.log(p3 + EPS)) / NC2

    e1_ref[...] = _topk5(sc1t_ref[...], NC1)
    e2_ref[...] = _topk5(sc2t, NC2)
    scal_ref[...] = jnp.concatenate(
        [_scal(ent1), _scal(mc1), _scal(o1), _scal(ent2), _scal(mc2),
         _scal(o2), _scal(ent3), jnp.zeros((1, 121), jnp.float32)], axis=1)


def kernel(x, edge_index, edge_weight, pos, subgraph_assignment, emb_table,
           gn_weight, gn_bias, gn_alpha, W1, b1, bn0_w, bn0_b, mlp1_W, mlp1_b,
           W2, b2, bn1_w, bn1_b, mlp2_W, mlp2_b, W3, b3, bn2_w, bn2_b,
           mlp3_W, mlp3_b):
    f32 = jnp.float32
    src = edge_index[0].reshape(N_EDGES // _C, _C)
    dst = edge_index[1].reshape(N_EDGES // _C, _C)
    ew2 = edge_weight.reshape(N_EDGES // _C, _C)
    x_col = x.astype(jnp.int32).reshape(N_NODES, 1)
    emb_pad = jnp.pad(emb_table, ((0, D_IN - emb_table.shape[0]), (0, 0)))
    row = lambda v: v.reshape(1, -1).astype(f32)

    tc1 = pl.pallas_call(
        _tc1_body,
        out_shape=jax.ShapeDtypeStruct((N_NODES, H1), f32))
    h_pre = tc1(x_col, emb_pad, row(gn_weight), row(gn_bias), row(gn_alpha),
                W1, row(b1))

    acc1, deg_parts = _make_edge_accum(H1, True)(src, dst, ew2, h_pre)

    tc2 = pl.pallas_call(
        _tc2_body,
        out_shape=[
            jax.ShapeDtypeStruct((N_NODES, _P1W), f32),
            jax.ShapeDtypeStruct((NC1, H1), f32),
            jax.ShapeDtypeStruct((NSUBG, NC1), f32),
            jax.ShapeDtypeStruct((1, 128), f32),
        ])
    p1pad, out1, sc1t, misc = tc2(acc1, deg_parts, subgraph_assignment,
                                  mlp1_W, row(mlp1_b), row(bn0_w),
                                  row(bn0_b))

    acc2, = _make_edge_accum(_P1W, False)(src, dst, ew2, p1pad)

    tc3 = pl.pallas_call(
        _tc3_body,
        out_shape=[
            jax.ShapeDtypeStruct((NSUBG, 8), f32),
            jax.ShapeDtypeStruct((NSUBG, 8), f32),
            jax.ShapeDtypeStruct((1, 128), f32),
        ])
    e1, e2, scal = tc3(acc2, p1pad, out1, sc1t, misc,
                       W2, row(b2), row(bn1_w), row(bn1_b), mlp2_W,
                       row(mlp2_b), W3, row(b3), row(bn2_w), row(bn2_b),
                       mlp3_W, row(mlp3_b))

    return jnp.concatenate(
        [e1[:, :5].ravel(), e2[:, :5].ravel(), scal[0, :7]])


# speedup vs baseline: 1.0136x; 1.0136x over previous
import jax
import jax.numpy as jnp
from jax import lax
from jax.experimental import pallas as pl
from jax.experimental.pallas import tpu as pltpu
from jax.experimental.pallas import tpu_sc as plsc

N_NODES = 4096
N_EDGES = 65536
D_IN = 128
H1 = 128
H2 = 128
NC1, NC2, NC3 = 100, 70, 50
NSUBG = 64
EPS = 1e-15

_L = 16
_NCORE = 2
_NSUB = 16
_NW = _NCORE * _NSUB
_C = 128
_P1W = 128
_DROWS = 32


def _make_edge_accum(d_in, with_w):
    ept = N_EDGES // _NW
    nchunk = ept // _C
    stripe = N_NODES // _NSUB
    mesh = plsc.VectorSubcoreMesh(core_axis_name="c", subcore_axis_name="s")

    nrow = ept // _C

    def body(src_hbm, dst_hbm, ew_hbm, val_hbm, *rest):
        if with_w:
            (out_hbm, outdeg_hbm, src_v, dst_v, w_v, g0_v, g1_v, g2_v,
             deg_v, acc, sem_g, sem_s) = rest
        else:
            (out_hbm, src_v, dst_v, w_v, g0_v, g1_v, g2_v, acc,
             sem_g, sem_s) = rest
        gbufs = (g0_v, g1_v, g2_v)
        f0_v = g0_v
        cid = lax.axis_index("c")
        sid = lax.axis_index("s")
        wid = cid * _NSUB + sid
        zeros16 = jnp.zeros((_L,), jnp.float32)
        lane = lax.broadcasted_iota(jnp.int32, (_L,), 0)
        mask0 = lane == 0

        pltpu.sync_copy(src_hbm.at[pl.ds(wid * nrow, nrow)], src_v)
        pltpu.sync_copy(dst_hbm.at[pl.ds(wid * nrow, nrow)], dst_v)
        pltpu.sync_copy(ew_hbm.at[pl.ds(wid * nrow, nrow)], w_v)

        def zero_row(j, carry):
            for q in range(d_in // _L):
                f0_v[j, pl.ds(q * _L, _L)] = zeros16
            return carry

        lax.fori_loop(0, _C, zero_row, 0)
        for t in range(stripe // _C):
            pltpu.sync_copy(f0_v,
                            acc.at[pl.ds(sid * stripe + t * _C, _C)])
        if with_w:
            def zero_deg(j, carry):
                for q in range(128 // _L):
                    deg_v[j, pl.ds(q * _L, _L)] = zeros16
                return carry

            lax.fori_loop(0, _DROWS, zero_deg, 0)

        nbuf = 3
        for j in range(min(nbuf, nchunk)):
            pltpu.async_copy(val_hbm.at[dst_v.at[j]], gbufs[j], sem_g.at[j])
        plsc.subcore_barrier()

        for kk in range(nchunk):
            b = kk % nbuf
            gbuf = gbufs[b]
            fbuf = gbuf
            pltpu.make_async_copy(val_hbm.at[dst_v.at[kk]], gbuf,
                                  sem_g.at[b]).wait()
            if kk >= 1 and kk + nbuf - 1 < nchunk:
                bb = (kk + nbuf - 1) % nbuf
                pltpu.make_async_copy(gbufs[bb], acc.at[src_v.at[kk - 1]],
                                      sem_s.at[bb]).wait()
                pltpu.async_copy(val_hbm.at[dst_v.at[kk + nbuf - 1]],
                                 gbufs[bb], sem_g.at[bb])

            kb = jnp.full((_L,), kk, jnp.int32)

            @plsc.parallel_loop(0, _C, unroll=16)
            def scale_row(e):
                eb = jnp.full((_L,), e, jnp.int32)
                wbc = plsc.load_gather(w_v, [kb, eb])
                for q in range(d_in // _L):
                    fbuf[e, pl.ds(q * _L, _L)] = (
                        gbuf[e, pl.ds(q * _L, _L)] * wbc)
                if with_w:
                    sbc = plsc.load_gather(src_v, [kb, eb])
                    plsc.addupdate_scatter(
                        deg_v,
                        [lax.shift_right_logical(sbc, 7),
                         jnp.bitwise_and(sbc, 127)],
                        wbc, mask=mask0)
            pltpu.async_copy(fbuf, acc.at[src_v.at[kk]], sem_s.at[b],
                             add=True)

        for j in range(max(nchunk - nbuf, 1), nchunk):
            pltpu.make_async_copy(gbufs[j % nbuf], acc.at[src_v.at[j]],
                                  sem_s.at[j % nbuf]).wait()
        plsc.subcore_barrier()
        for t in range(stripe // _C):
            r0 = sid * stripe + t * _C
            pltpu.sync_copy(acc.at[pl.ds(r0, _C)],
                            out_hbm.at[pl.ds(cid * N_NODES + r0, _C)])
        if with_w:
            pltpu.sync_copy(deg_v, outdeg_hbm.at[pl.ds(wid * _DROWS, _DROWS)])

    out_type = [jax.ShapeDtypeStruct((_NCORE * N_NODES, d_in), jnp.float32)]
    scratch = [
        pltpu.VMEM((nrow, _C), jnp.int32),
        pltpu.VMEM((nrow, _C), jnp.int32),
        pltpu.VMEM((nrow, _C), jnp.float32),
        pltpu.VMEM((_C, d_in), jnp.float32),
        pltpu.VMEM((_C, d_in), jnp.float32),
        pltpu.VMEM((_C, d_in), jnp.float32),
    ]
    if with_w:
        out_type.append(
            jax.ShapeDtypeStruct((_NW * _DROWS, 128), jnp.float32))
        scratch.append(pltpu.VMEM((_DROWS, 128), jnp.float32))
    scratch += [
        pltpu.VMEM_SHARED((N_NODES, d_in), jnp.float32),
        pltpu.SemaphoreType.DMA((3,)),
        pltpu.SemaphoreType.DMA((3,)),
    ]
    return pl.kernel(
        body,
        mesh=mesh,
        compiler_params=pltpu.CompilerParams(needs_layout_passes=False),
        out_type=out_type,
        scratch_types=scratch,
    )


def _tc1_body(x_ref, emb_ref, gnw_ref, gnb_ref, gna_ref, w1_ref, b1_ref,
              out_ref):
    xv = x_ref[...]
    col = lax.broadcasted_iota(jnp.int32, (N_NODES, D_IN), 1)
    onehot = (col == xv).astype(jnp.float32)
    x1 = jnp.dot(onehot, emb_ref[...], preferred_element_type=jnp.float32)
    mean = jnp.mean(x1, axis=0, keepdims=True)
    cent = x1 - gna_ref[...] * mean
    var = jnp.mean(cent * cent, axis=0, keepdims=True)
    x1n = gnw_ref[...] * cent / jnp.sqrt(var + 1e-5) + gnb_ref[...]
    h = jnp.dot(x1n, w1_ref[...], preferred_element_type=jnp.float32)
    out_ref[...] = jnp.maximum(h + b1_ref[...], 0.0)


def _scal(v):
    return jnp.full((1, 1), v, jnp.float32)


def _tc2_body(acc_ref, deg_ref, sa_ref, m1w_ref, m1b_ref, bnw_ref,
              bnb_ref, p1_ref, out1_ref, sc1t_ref, misc_ref):
    swh = acc_ref[0:N_NODES, :] + acc_ref[N_NODES:2 * N_NODES, :]
    degsum = deg_ref[0:_DROWS, :]
    for t in range(1, _NW):
        degsum = degsum + deg_ref[t * _DROWS:(t + 1) * _DROWS, :]
    ri32 = lax.broadcasted_iota(jnp.int32, (N_NODES, _DROWS), 0)
    ci32 = lax.broadcasted_iota(jnp.int32, (N_NODES, _DROWS), 1)
    a1 = (ci32 == lax.shift_right_logical(ri32, 7)).astype(jnp.float32)
    t128 = jnp.dot(a1, degsum, preferred_element_type=jnp.float32)
    ri = lax.broadcasted_iota(jnp.int32, (N_NODES, 128), 0)
    ci = lax.broadcasted_iota(jnp.int32, (N_NODES, 128), 1)
    msel = (ci == jnp.bitwise_and(ri, 127)).astype(jnp.float32)
    deg = jnp.sum(t128 * msel, axis=1, keepdims=True)
    degc = jnp.where(deg < 0.5, deg + 1.0, deg)
    conv = swh / degc
    mean = jnp.mean(conv, axis=0, keepdims=True)
    var = jnp.mean(conv * conv, axis=0, keepdims=True) - mean * mean
    h = jnp.maximum(
        (conv - mean) / jnp.sqrt(var + 1e-5) * bnw_ref[...] + bnb_ref[...],
        0.0)
    s1 = jnp.dot(h, m1w_ref[...], preferred_element_type=jnp.float32) \
        + m1b_ref[...]
    mx = jnp.max(s1, axis=1, keepdims=True)
    ex = jnp.exp(s1 - mx)
    p1 = ex / jnp.sum(ex, axis=1, keepdims=True)
    ent1 = jnp.sum(-p1 * jnp.log(p1 + EPS)) / N_NODES
    csum = jnp.sum(p1, axis=0, keepdims=True)
    inv = 1.0 / jnp.maximum(csum, 1e-12)
    sc1t = lax.dot_general(sa_ref[...], p1,
                           (((1,), (0,)), ((), ()))) * inv
    out1 = lax.dot_general(p1, h, (((0,), (0,)), ((), ())))
    ss = lax.dot_general(p1, p1, (((0,), (0,)), ((), ())))
    ssf = jnp.sqrt(jnp.sum(ss * ss))
    r_i = lax.broadcasted_iota(jnp.int32, (NC1, NC1), 0)
    c_i = lax.broadcasted_iota(jnp.int32, (NC1, NC1), 1)
    eye1 = (r_i == c_i).astype(jnp.float32)
    o1 = jnp.sqrt(jnp.sum((ss / ssf - eye1 * (1.0 / jnp.sqrt(float(NC1))))**2))
    den1 = jnp.sum(deg * jnp.sum(p1 * p1, axis=1, keepdims=True))
    p1_ref[...] = jnp.concatenate(
        [p1, jnp.zeros((N_NODES, _P1W - NC1), jnp.float32)], axis=1)
    out1_ref[...] = out1
    sc1t_ref[...] = sc1t
    misc_ref[...] = jnp.concatenate(
        [_scal(ent1), _scal(den1), _scal(o1),
         jnp.zeros((1, 125), jnp.float32)], axis=1)


def _eye(n):
    r_i = lax.broadcasted_iota(jnp.int32, (n, n), 0)
    c_i = lax.broadcasted_iota(jnp.int32, (n, n), 1)
    return (r_i == c_i).astype(jnp.float32)


def _row_of(colvec, eye):
    return lax.dot_general(colvec, eye, (((0,), (0,)), ((), ())))


def _topk5(m, ncols):
    iot = lax.broadcasted_iota(jnp.int32, m.shape, 1)
    cols = []
    cur = m
    for _ in range(5):
        mxv = jnp.max(cur, axis=1, keepdims=True)
        cols.append(mxv)
        pos = jnp.min(jnp.where(cur == mxv, iot, ncols), axis=1,
                      keepdims=True)
        cur = jnp.where(iot == pos, -1e30, cur)
    return jnp.concatenate(
        cols + [jnp.zeros((m.shape[0], 3), jnp.float32)], axis=1)


def _tc3_body(acc_ref, p1_ref, out1_ref, sc1t_ref, misc_ref,
              w2_ref, b2_ref, bn1w_ref, bn1b_ref, m2w_ref, m2b_ref,
              w3_ref, b3_ref, bn2w_ref, bn2b_ref, m3w_ref, m3b_ref,
              e1_ref, e2_ref, scal_ref):
    As = acc_ref[0:N_NODES, :] + acc_ref[N_NODES:2 * N_NODES, :]
    p1 = p1_ref[...]
    oadj_t = lax.dot_general(p1, As, (((0,), (0,)), ((), ())))
    oadj = oadj_t[0:NC1, 0:NC1]
    ent1 = misc_ref[0, 0]
    den1 = misc_ref[0, 1]
    o1 = misc_ref[0, 2]
    eye1 = _eye(NC1)
    mc1 = -(jnp.sum(oadj * eye1) / den1)
    oadj = oadj * (1.0 - eye1)
    d1 = jnp.sqrt(jnp.sum(oadj, axis=1, keepdims=True)) + EPS
    oadj1 = oadj / d1 / _row_of(d1, eye1)

    deg2 = jnp.sum(oadj1, axis=1, keepdims=True)
    deg2c = jnp.where(deg2 < 0.5, deg2 + 1.0, deg2)
    hh = jnp.maximum(
        jnp.dot(out1_ref[...], w2_ref[...],
                preferred_element_type=jnp.float32) + b2_ref[...], 0.0)
    h2c = jnp.dot(oadj1 / deg2c, hh, preferred_element_type=jnp.float32)
    m2 = jnp.mean(h2c, axis=0, keepdims=True)
    v2 = jnp.mean(h2c * h2c, axis=0, keepdims=True) - m2 * m2
    h2 = jnp.maximum(
        (h2c - m2) / jnp.sqrt(v2 + 1e-5) * bn1w_ref[...] + bn1b_ref[...], 0.0)
    s2 = jnp.dot(h2, m2w_ref[...], preferred_element_type=jnp.float32) \
        + m2b_ref[...]
    mx2 = jnp.max(s2, axis=1, keepdims=True)
    ex2 = jnp.exp(s2 - mx2)
    p2 = ex2 / jnp.sum(ex2, axis=1, keepdims=True)
    ent2 = jnp.sum(-p2 * jnp.log(p2 + EPS)) / NC1
    cs2 = jnp.sum(p2, axis=0, keepdims=True)
    sc2t = lax.dot_general(sc1t_ref[...], p2, (((1,), (0,)), ((), ()))) \
        * (1.0 / jnp.maximum(cs2, 1e-12))

    out2 = lax.dot_general(p2, h2, (((0,), (0,)), ((), ())))
    ap2 = jnp.dot(oadj1, p2, preferred_element_type=jnp.float32)
    oadj2t = lax.dot_general(p2, ap2, (((0,), (0,)), ((), ())))
    den2 = jnp.sum(deg2 * jnp.sum(p2 * p2, axis=1, keepdims=True))
    eye2 = _eye(NC2)
    mc2 = -(jnp.sum(oadj2t * eye2) / den2)
    ss2 = lax.dot_general(p2, p2, (((0,), (0,)), ((), ())))
    ssf2 = jnp.sqrt(jnp.sum(ss2 * ss2))
    o2 = jnp.sqrt(jnp.sum(
        (ss2 / ssf2 - eye2 * (1.0 / jnp.sqrt(float(NC2))))**2))
    oadj2 = oadj2t * (1.0 - eye2)
    dd = jnp.sqrt(jnp.sum(oadj2, axis=1, keepdims=True)) + EPS
    oadj2n = oadj2 / dd / _row_of(dd, eye2)

    deg3 = jnp.sum(oadj2n, axis=1, keepdims=True)
    deg3c = jnp.where(deg3 < 0.5, deg3 + 1.0, deg3)
    hh3 = jnp.maximum(
        jnp.dot(out2, w3_ref[...],
                preferred_element_type=jnp.float32) + b3_ref[...], 0.0)
    h3c = jnp.dot(oadj2n / deg3c, hh3, preferred_element_type=jnp.float32)
    m3 = jnp.mean(h3c, axis=0, keepdims=True)
    v3 = jnp.mean(h3c * h3c, axis=0, keepdims=True) - m3 * m3
    h3 = jnp.maximum(
        (h3c - m3) / jnp.sqrt(v3 + 1e-5) * bn2w_ref[...] + bn2b_ref[...], 0.0)
    s3 = jnp.dot(h3, m3w_ref[...], preferred_element_type=jnp.float32) \
        + m3b_ref[...]
    mx3 = jnp.max(s3, axis=1, keepdims=True)
    ex3 = jnp.exp(s3 - mx3)
    p3 = ex3 / jnp.sum(ex3, axis=1, keepdims=True)
    ent3 = jnp.sum(-p3 * jnp.log(p3 + EPS)) / NC2

    e1_ref[...] = _topk5(sc1t_ref[...], NC1)
    e2_ref[...] = _topk5(sc2t, NC2)
    scal_ref[...] = jnp.concatenate(
        [_scal(ent1), _scal(mc1), _scal(o1), _scal(ent2), _scal(mc2),
         _scal(o2), _scal(ent3), jnp.zeros((1, 121), jnp.float32)], axis=1)


def kernel(x, edge_index, edge_weight, pos, subgraph_assignment, emb_table,
           gn_weight, gn_bias, gn_alpha, W1, b1, bn0_w, bn0_b, mlp1_W, mlp1_b,
           W2, b2, bn1_w, bn1_b, mlp2_W, mlp2_b, W3, b3, bn2_w, bn2_b,
           mlp3_W, mlp3_b):
    f32 = jnp.float32
    src = edge_index[0].reshape(N_EDGES // _C, _C)
    dst = edge_index[1].reshape(N_EDGES // _C, _C)
    ew2 = edge_weight.reshape(N_EDGES // _C, _C)
    x_col = x.astype(jnp.int32).reshape(N_NODES, 1)
    emb_pad = jnp.pad(emb_table, ((0, D_IN - emb_table.shape[0]), (0, 0)))
    row = lambda v: v.reshape(1, -1).astype(f32)

    tc1 = pl.pallas_call(
        _tc1_body,
        out_shape=jax.ShapeDtypeStruct((N_NODES, H1), f32))
    h_pre = tc1(x_col, emb_pad, row(gn_weight), row(gn_bias), row(gn_alpha),
                W1, row(b1))

    acc1, deg_parts = _make_edge_accum(H1, True)(src, dst, ew2, h_pre)

    tc2 = pl.pallas_call(
        _tc2_body,
        out_shape=[
            jax.ShapeDtypeStruct((N_NODES, _P1W), f32),
            jax.ShapeDtypeStruct((NC1, H1), f32),
            jax.ShapeDtypeStruct((NSUBG, NC1), f32),
            jax.ShapeDtypeStruct((1, 128), f32),
        ])
    p1pad, out1, sc1t, misc = tc2(acc1, deg_parts, subgraph_assignment,
                                  mlp1_W, row(mlp1_b), row(bn0_w),
                                  row(bn0_b))

    acc2, = _make_edge_accum(_P1W, False)(src, dst, ew2, p1pad)

    tc3 = pl.pallas_call(
        _tc3_body,
        out_shape=[
            jax.ShapeDtypeStruct((NSUBG, 8), f32),
            jax.ShapeDtypeStruct((NSUBG, 8), f32),
            jax.ShapeDtypeStruct((1, 128), f32),
        ])
    e1, e2, scal = tc3(acc2, p1pad, out1, sc1t, misc,
                       W2, row(b2), row(bn1_w), row(bn1_b), mlp2_W,
                       row(mlp2_b), W3, row(b3), row(bn2_w), row(bn2_b),
                       mlp3_W, row(mlp3_b))

    return jnp.concatenate(
        [e1[:, :5].ravel(), e2[:, :5].ravel(), scal[0, :7]])
